# baseline (speedup 1.0000x reference)
import jax
import jax.numpy as jnp
from jax import lax
from jax.experimental import pallas as pl
from jax.experimental.pallas import tpu as pltpu
from jax.experimental.pallas import tpu_sc as plsc

BATCH = 4096
HIST = 200
EMBED = 64
L = 16
HPAD = 208
HHALF = 104
NC, NS = 2, 16
NW = NC * NS
BPW = BATCH // NW
KREG = EMBED // L


def _rsqrt_newton(p):
    bits = plsc.bitcast(p, jnp.int32)
    bits = jnp.full((L,), 0x5F3759DF, jnp.int32) - (bits >> 1)
    y = plsc.bitcast(bits, jnp.float32)
    for _ in range(3):
        y = y * (1.5 - 0.5 * p * y * y)
    return y


def _lane_sum(red_v, x):
    lane = lax.iota(jnp.int32, L)
    for s in (8, 4, 2, 1):
        red_v[...] = x
        x = x + plsc.load_gather(red_v, [lane ^ s])
    return x


def _body(seq1_hbm, seq2_hbm, table_hbm, sim_hbm, vec1_hbm, vec2_hbm,
          idx_v, rows_v, vec1_v, vec2_v, sim_v, red_v, sem):
    wid = lax.axis_index("s") * NC + lax.axis_index("c")
    base = wid * BPW

    zf = jnp.zeros((L,), jnp.float32)
    idx_v[pl.ds(HPAD - L, L)] = jnp.zeros((L,), jnp.int32)

    def encode(seq_hbm, i, out_v):
        off = pl.multiple_of((base + i) * HIST, 8)
        pltpu.sync_copy(seq_hbm.at[pl.ds(off, HIST)], idx_v.at[pl.ds(0, HIST)])
        c1 = pltpu.async_copy(table_hbm.at[idx_v.at[pl.ds(0, HHALF)]],
                              rows_v.at[pl.ds(0, HHALF)], sem)
        c2 = pltpu.async_copy(table_hbm.at[idx_v.at[pl.ds(HHALF, HHALF)]],
                              rows_v.at[pl.ds(HHALF, HHALF)], sem)
        cnt = zf
        for j in range(HPAD // L):
            v = idx_v[pl.ds(j * L, L)]
            cnt = cnt + jnp.where(v != 0, 1.0, 0.0).astype(jnp.float32)
        c1.wait()
        c2.wait()

        def red(j, acc):
            return tuple(acc[k] + rows_v[j, pl.ds(k * L, L)]
                         for k in range(KREG))

        acc = lax.fori_loop(0, HPAD, red, (zf,) * KREG)
        denom = jnp.maximum(_lane_sum(red_v, cnt), 1e-9)
        vs = tuple(acc[k] / denom for k in range(KREG))
        for k in range(KREG):
            out_v[i, pl.ds(k * L, L)] = vs[k]
        return vs

    def row(i, carry):
        v1 = encode(seq1_hbm, i, vec1_v)
        v2 = encode(seq2_hbm, i, vec2_v)
        dot, n1, n2 = zf, zf, zf
        for k in range(KREG):
            dot = dot + v1[k] * v2[k]
            n1 = n1 + v1[k] * v1[k]
            n2 = n2 + v2[k] * v2[k]
        p = jnp.maximum(_lane_sum(red_v, n1) * _lane_sum(red_v, n2), 1e-16)
        sim = _lane_sum(red_v, dot) * _rsqrt_newton(p)
        lane = lax.iota(jnp.int32, L)
        plsc.store_scatter(sim_v, [jnp.full((L,), i, jnp.int32)],
                           jnp.full((L,), sim, jnp.float32), mask=lane == 0)
        return carry

    lax.fori_loop(0, BPW, row, 0)

    pltpu.sync_copy(sim_v, sim_hbm.at[pl.ds(base, BPW)])
    pltpu.sync_copy(vec1_v, vec1_hbm.at[pl.ds(base, BPW)])
    pltpu.sync_copy(vec2_v, vec2_hbm.at[pl.ds(base, BPW)])


def kernel(seq1, seq2, table):
    f = pl.kernel(
        _body,
        out_type=(
            jax.ShapeDtypeStruct((BATCH,), jnp.float32),
            jax.ShapeDtypeStruct((BATCH, EMBED), jnp.float32),
            jax.ShapeDtypeStruct((BATCH, EMBED), jnp.float32),
        ),
        mesh=plsc.VectorSubcoreMesh(core_axis_name="c", subcore_axis_name="s"),
        compiler_params=pltpu.CompilerParams(needs_layout_passes=False,
                                             use_tc_tiling_on_sc=False),
        scratch_types=[
            pltpu.VMEM((HPAD,), jnp.int32),
            pltpu.VMEM((HPAD, EMBED), jnp.float32),
            pltpu.VMEM((BPW, EMBED), jnp.float32),
            pltpu.VMEM((BPW, EMBED), jnp.float32),
            pltpu.VMEM((BPW,), jnp.float32),
            pltpu.VMEM((L,), jnp.float32),
            pltpu.SemaphoreType.DMA,
        ],
    )
    return f(seq1.astype(jnp.int32).reshape(-1),
             seq2.astype(jnp.int32).reshape(-1), table)
